# baseline (speedup 1.0000x reference)
import functools

import jax
import jax.numpy as jnp
from jax import lax
from jax.experimental import pallas as pl
from jax.experimental.pallas import tpu as pltpu
from jax.experimental.pallas import tpu_sc as plsc

BATCH = 16384
D = 64
NC, NS = 2, 16
NW = NC * NS
B_PER_W = BATCH // NW
CHUNK = 128
N_CHUNKS = B_PER_W // CHUNK

BLK = 1024
N_BLK = BATCH // BLK


def _gather_sc(uidx2d, midx2d, user_table, movie_table):
    mesh = plsc.VectorSubcoreMesh(core_axis_name="c", subcore_axis_name="s")

    @functools.partial(
        pl.kernel,
        mesh=mesh,
        compiler_params=pltpu.CompilerParams(use_tc_tiling_on_sc=False),
        out_type=jax.ShapeDtypeStruct((BATCH, 2 * D), jnp.float32),
        scratch_types=[
            pltpu.VMEM((N_CHUNKS, CHUNK), jnp.int32),
            pltpu.VMEM((N_CHUNKS, CHUNK), jnp.int32),
            pltpu.VMEM((B_PER_W, D), jnp.float32),
            pltpu.VMEM((B_PER_W, D), jnp.float32),
            pltpu.SemaphoreType.DMA,
            pltpu.SemaphoreType.DMA,
        ],
    )
    def gather_kernel(uidx_hbm, midx_hbm, utab_hbm, mtab_hbm,
                      x_hbm,
                      uidx_v, midx_v, urows_v, mrows_v, usem, msem):
        wid = lax.axis_index("s") * NC + lax.axis_index("c")
        pltpu.sync_copy(uidx_hbm.at[pl.ds(wid * N_CHUNKS, N_CHUNKS)], uidx_v)
        pltpu.sync_copy(midx_hbm.at[pl.ds(wid * N_CHUNKS, N_CHUNKS)], midx_v)
        copies = []
        for j in range(N_CHUNKS):
            copies.append(pltpu.async_copy(
                utab_hbm.at[uidx_v.at[j]],
                urows_v.at[pl.ds(j * CHUNK, CHUNK)], usem))
            copies.append(pltpu.async_copy(
                mtab_hbm.at[midx_v.at[j]],
                mrows_v.at[pl.ds(j * CHUNK, CHUNK)], msem))
        for c in copies:
            c.wait()
        base = wid * B_PER_W
        pltpu.sync_copy(urows_v, x_hbm.at[pl.ds(base, B_PER_W), pl.ds(0, D)])
        pltpu.sync_copy(mrows_v, x_hbm.at[pl.ds(base, B_PER_W), pl.ds(D, D)])

    return gather_kernel(uidx2d, midx2d, user_table, movie_table)


def _mlp_body(x_ref, w1t_ref, b1_ref, w2_ref, b2_ref, out_ref):
    h = jnp.dot(x_ref[...], w1t_ref[...], preferred_element_type=jnp.float32)
    h = jnp.maximum(h + b1_ref[...], 0.0)
    row = lax.dot_general(w2_ref[...], h, (((1,), (1,)), ((), ())),
                          preferred_element_type=jnp.float32)
    out_ref[...] = (row + b2_ref[0, 0]).reshape(1, 1, BLK)


def kernel(user_idx, movie_idx, user_table, movie_table, W1, b1, W2, b2):
    uidx2d = user_idx.astype(jnp.int32).reshape(NW * N_CHUNKS, CHUNK)
    midx2d = movie_idx.astype(jnp.int32).reshape(NW * N_CHUNKS, CHUNK)
    x = _gather_sc(uidx2d, midx2d, user_table, movie_table)
    w1t = W1.T
    out2d = pl.pallas_call(
        _mlp_body,
        grid=(N_BLK,),
        in_specs=[
            pl.BlockSpec((BLK, 2 * D), lambda i: (i, 0)),
            pl.BlockSpec((2 * D, D), lambda i: (0, 0)),
            pl.BlockSpec((1, D), lambda i: (0, 0)),
            pl.BlockSpec((1, D), lambda i: (0, 0)),
            pl.BlockSpec((1, 1), lambda i: (0, 0)),
        ],
        out_specs=pl.BlockSpec((1, 1, BLK), lambda i: (i, 0, 0)),
        out_shape=jax.ShapeDtypeStruct((N_BLK, 1, BLK), jnp.float32),
    )(x, w1t, b1.reshape(1, D), W2, b2.reshape(1, 1))
    return out2d.reshape(BATCH)
